# Initial kernel scaffold; baseline (speedup 1.0000x reference)
import jax
import jax.numpy as jnp
from jax.experimental import pallas as pl


def kernel(node_features, edge_index, edge_weight, W1, b1, W2, b2, W3, b3, in_proj_w, in_proj_b, out_proj_w, out_proj_b):
    raise NotImplementedError("write your pallas kernel here")


# speedup vs baseline: 4.2880x; 4.2880x over previous
import functools

import jax
import jax.numpy as jnp
import numpy as np
from jax import lax
from jax.experimental import pallas as pl
from jax.experimental.pallas import tpu as pltpu
from jax.experimental.pallas import tpu_sc as plsc

_N = 10000
_E = 320000
_DH = 128
_DOUT = 64
_H = 4
_DHEAD = 32

_NC, _NS, _L = 2, 16, 16
_NW = _NC * _NS
_CHUNK = 128
_EPT = 10240
_EPAD = _NW * _EPT
_NCHUNK = _EPT // _CHUNK
_NACC = 10240
_RPT = _NACC // _NS

_BN = 1000
_BQ = 400
_BK = 1000
_F32 = jnp.float32


def _sc_mesh():
    return plsc.VectorSubcoreMesh(
        core_axis_name="c", subcore_axis_name="s",
        num_cores=_NC, num_subcores=_NS)


def _make_sc_conv(d):
    @functools.partial(
        pl.kernel,
        mesh=_sc_mesh(),
        out_type=jax.ShapeDtypeStruct((_NC, _NACC, d), _F32),
        scratch_types=[
            pltpu.VMEM((_CHUNK,), jnp.int32),
            pltpu.VMEM((_CHUNK,), jnp.int32),
            pltpu.VMEM((_CHUNK,), _F32),
            pltpu.VMEM((_CHUNK, d), _F32),
            pltpu.VMEM_SHARED((_NACC, d), _F32),
            pltpu.SemaphoreType.DMA,
        ],
        name=f"sc_gcn_scatter_d{d}",
        compiler_params=pltpu.CompilerParams(use_tc_tiling_on_sc=(d % 128 == 0)),
    )
    def conv(h_hbm, src_hbm, dst_hbm, w_hbm, zeros_hbm, out_hbm,
             src_v, dst_v, w_v, rows_v, acc, sem):
        cid = lax.axis_index("c")
        sid = lax.axis_index("s")
        wid = sid * _NC + cid
        r0 = sid * _RPT
        pltpu.sync_copy(zeros_hbm.at[pl.ds(r0, _RPT)], acc.at[pl.ds(r0, _RPT)])
        plsc.subcore_barrier()
        base = wid * _EPT

        def chunk(c, carry):
            off = base + c * _CHUNK
            pltpu.sync_copy(src_hbm.at[pl.ds(off, _CHUNK)], src_v)
            pltpu.sync_copy(dst_hbm.at[pl.ds(off, _CHUNK)], dst_v)
            pltpu.sync_copy(w_hbm.at[pl.ds(off, _CHUNK)], w_v)
            pltpu.async_copy(h_hbm.at[src_v], rows_v, sem).wait()

            def grp(g, cc):
                wv = w_v[pl.ds(g * _L, _L)]
                for i in range(_L):
                    nv = jnp.zeros((_L,), _F32) + wv[i]
                    r = g * _L + i
                    for j in range(d // _L):
                        sl = pl.ds(j * _L, _L)
                        rows_v[r, sl] = rows_v[r, sl] * nv
                return cc

            lax.fori_loop(0, _CHUNK // _L, grp, 0, unroll=False)
            pltpu.sync_copy(rows_v, acc.at[dst_v], add=True)
            return carry

        lax.fori_loop(0, _NCHUNK, chunk, 0, unroll=False)
        plsc.subcore_barrier()
        pltpu.sync_copy(acc.at[pl.ds(r0, _RPT)],
                        out_hbm.at[cid, pl.ds(r0, _RPT)])

    return conv


@functools.partial(
    pl.kernel,
    mesh=_sc_mesh(),
    out_type=jax.ShapeDtypeStruct((_NC, _NACC), _F32),
    scratch_types=[
        pltpu.VMEM((_CHUNK,), jnp.int32),
        pltpu.VMEM((_CHUNK,), _F32),
        pltpu.VMEM_SHARED((_NACC,), _F32),
    ],
    name="sc_gcn_degree",
)
def _sc_degree(dst_hbm, w_hbm, zeros_hbm, out_hbm, dst_v, w_v, acc):
    cid = lax.axis_index("c")
    sid = lax.axis_index("s")
    wid = sid * _NC + cid
    r0 = sid * _RPT
    pltpu.sync_copy(zeros_hbm.at[pl.ds(r0, _RPT)], acc.at[pl.ds(r0, _RPT)])
    plsc.subcore_barrier()
    base = wid * _EPT

    def chunk(c, carry):
        off = base + c * _CHUNK
        pltpu.sync_copy(dst_hbm.at[pl.ds(off, _CHUNK)], dst_v)
        pltpu.sync_copy(w_hbm.at[pl.ds(off, _CHUNK)], w_v)
        pltpu.sync_copy(w_v, acc.at[dst_v], add=True)
        return carry

    lax.fori_loop(0, _NCHUNK, chunk, 0, unroll=False)
    plsc.subcore_barrier()
    pltpu.sync_copy(acc.at[pl.ds(r0, _RPT)], out_hbm.at[cid, pl.ds(r0, _RPT)])


def _dinv_body(dp_ref, dinv_ref):
    dp = dp_ref[...]
    dinv_ref[...] = lax.rsqrt(dp[0] + dp[1] + 1.0)


_dinv = pl.pallas_call(
    _dinv_body,
    grid=(1,),
    in_specs=[pl.BlockSpec((2, _NACC // 128, 128), lambda i: (0, 0, 0))],
    out_specs=pl.BlockSpec((_NACC // 128, 128), lambda i: (0, 0)),
    out_shape=jax.ShapeDtypeStruct((_NACC // 128, 128), _F32),
)


def _prep_body(dinv_ref, nf_ref, w1t_ref, hp_ref):
    hp_ref[...] = dinv_ref[...] * jnp.dot(nf_ref[...], w1t_ref[...],
                                          preferred_element_type=_F32)


_prep = pl.pallas_call(
    _prep_body,
    grid=(_N // _BN,),
    in_specs=[
        pl.BlockSpec((_BN, 1), lambda i: (i, 0)),
        pl.BlockSpec((_BN, _DH), lambda i: (i, 0)),
        pl.BlockSpec((_DH, _DH), lambda i: (0, 0)),
    ],
    out_specs=pl.BlockSpec((_BN, _DH), lambda i: (i, 0)),
    out_shape=jax.ShapeDtypeStruct((_N, _DH), _F32),
)


def _qkv_body(parts_ref, hp_ref, dinv_ref, bc_ref, wt_ref, inb_ref, qkv_ref):
    p = parts_ref[...]
    x = dinv_ref[...] * (p[0] + p[1] + hp_ref[...]) + bc_ref[...]
    qkv_ref[...] = jnp.dot(x, wt_ref[...],
                           preferred_element_type=_F32) + inb_ref[...]


_qkv = pl.pallas_call(
    _qkv_body,
    grid=(_N // _BN,),
    in_specs=[
        pl.BlockSpec((2, _BN, _DH), lambda i: (0, i, 0)),
        pl.BlockSpec((_BN, _DH), lambda i: (i, 0)),
        pl.BlockSpec((_BN, 1), lambda i: (i, 0)),
        pl.BlockSpec((1, _DH), lambda i: (0, 0)),
        pl.BlockSpec((_DH, 3 * _DH), lambda i: (0, 0)),
        pl.BlockSpec((1, 3 * _DH), lambda i: (0, 0)),
    ],
    out_specs=pl.BlockSpec((_BN, 3 * _DH), lambda i: (i, 0)),
    out_shape=jax.ShapeDtypeStruct((_N, 3 * _DH), _F32),
)


def _attn_body(dnext, q_ref, k_ref, v_ref, dinv_ref, owt_ref, ob_ref,
               wnt_ref, out_ref):
    q = q_ref[...]
    scale = np.float32(1.0 / np.sqrt(_DHEAD))
    neg = np.float32(-1e30)
    ms = [jnp.full((_BQ, 1), neg, _F32) for _ in range(_H)]
    ls = [jnp.zeros((_BQ, 1), _F32) for _ in range(_H)]
    accs = [jnp.zeros((_BQ, _DHEAD), _F32) for _ in range(_H)]

    def kb_body(kb, carry):
        ms, ls, accs = carry
        kblk = k_ref[pl.ds(kb * _BK, _BK), :]
        vblk = v_ref[pl.ds(kb * _BK, _BK), :]
        nms, nls, naccs = [], [], []
        for h in range(_H):
            hs = slice(h * _DHEAD, (h + 1) * _DHEAD)
            qh = q[:, hs] * scale
            s = lax.dot_general(qh, kblk[:, hs], (((1,), (1,)), ((), ())),
                                preferred_element_type=_F32)
            m_new = jnp.maximum(ms[h], jnp.max(s, axis=1, keepdims=True))
            p = jnp.exp(s - m_new)
            corr = jnp.exp(ms[h] - m_new)
            l_new = ls[h] * corr + jnp.sum(p, axis=1, keepdims=True)
            a_new = accs[h] * corr + lax.dot_general(
                p, vblk[:, hs], (((1,), (0,)), ((), ())),
                preferred_element_type=_F32)
            nms.append(m_new)
            nls.append(l_new)
            naccs.append(a_new)
        return tuple(nms), tuple(nls), tuple(naccs)

    ms, ls, accs = lax.fori_loop(0, _N // _BK, kb_body,
                                 (tuple(ms), tuple(ls), tuple(accs)))
    o = jnp.concatenate([accs[h] / ls[h] for h in range(_H)], axis=1)
    t = jnp.maximum(jnp.dot(o, owt_ref[...], preferred_element_type=_F32)
                    + ob_ref[...], 0.0)
    out_ref[...] = dinv_ref[...] * jnp.dot(t, wnt_ref[...],
                                           preferred_element_type=_F32)


def _make_attn(dnext):
    return pl.pallas_call(
        functools.partial(_attn_body, dnext),
        grid=(_N // _BQ,),
        in_specs=[
            pl.BlockSpec((_BQ, _DH), lambda i: (i, 0)),
            pl.BlockSpec((_N, _DH), lambda i: (0, 1)),
            pl.BlockSpec((_N, _DH), lambda i: (0, 2)),
            pl.BlockSpec((_BQ, 1), lambda i: (i, 0)),
            pl.BlockSpec((_DH, _DH), lambda i: (0, 0)),
            pl.BlockSpec((1, _DH), lambda i: (0, 0)),
            pl.BlockSpec((_DH, dnext), lambda i: (0, 0)),
        ],
        out_specs=pl.BlockSpec((_BQ, dnext), lambda i: (i, 0)),
        out_shape=jax.ShapeDtypeStruct((_N, dnext), _F32),
    )


def _final_body(parts_ref, hp_ref, dinv_ref, b3_ref, out_ref):
    p = parts_ref[...]
    z = dinv_ref[...] * (p[0] + p[1] + hp_ref[...]) + b3_ref[...]
    out_ref[...] = jax.nn.sigmoid(z)


_final = pl.pallas_call(
    _final_body,
    grid=(_N // _BN,),
    in_specs=[
        pl.BlockSpec((2, _BN, _DOUT), lambda i: (0, i, 0)),
        pl.BlockSpec((_BN, _DOUT), lambda i: (i, 0)),
        pl.BlockSpec((_BN, 1), lambda i: (i, 0)),
        pl.BlockSpec((1, _DOUT), lambda i: (0, 0)),
    ],
    out_specs=pl.BlockSpec((_BN, _DOUT), lambda i: (i, 0)),
    out_shape=jax.ShapeDtypeStruct((_N, _DOUT), _F32),
)

_conv128 = _make_sc_conv(_DH)
_conv64 = _make_sc_conv(_DOUT)
_attn128 = _make_attn(_DH)
_attn64 = _make_attn(_DOUT)


def kernel(node_features, edge_index, edge_weight, W1, b1, W2, b2, W3, b3,
           in_proj_w, in_proj_b, out_proj_w, out_proj_b):
    src = edge_index[0]
    dst = edge_index[1]
    pad = _EPAD - _E
    zi = jnp.zeros((pad,), jnp.int32)
    src_p = jnp.concatenate([src, zi])
    dst_p = jnp.concatenate([dst, zi])
    w_p = jnp.concatenate([edge_weight, jnp.zeros((pad,), _F32)])

    w1t = W1.T
    w2t = W2.T
    w3t = W3.T
    inwt = in_proj_w.T
    owt = out_proj_w.T
    b1r = b1.reshape(1, -1)
    b2r = b2.reshape(1, -1)
    b3r = b3.reshape(1, -1)
    inbr = in_proj_b.reshape(1, -1)
    obr = out_proj_b.reshape(1, -1)

    z1 = jnp.zeros((_NACC,), _F32)
    z128 = jnp.zeros((_NACC, _DH), _F32)
    z64 = jnp.zeros((_NACC, _DOUT), _F32)

    degp = _sc_degree(dst_p, w_p, z1)
    dinv_lm = _dinv(degp.reshape(_NC, _NACC // 128, 128))
    dinv = dinv_lm.reshape(_NACC, 1)[:_N]
    hp1 = _prep(dinv, node_features, w1t)

    parts1 = _conv128(hp1, src_p, dst_p, w_p, z128)
    qkv1 = _qkv(parts1, hp1, dinv, b1r, inwt, inbr)
    hp2 = _attn128(qkv1, qkv1, qkv1, dinv, owt, obr, w2t)

    parts2 = _conv128(hp2, src_p, dst_p, w_p, z128)
    qkv2 = _qkv(parts2, hp2, dinv, b2r, inwt, inbr)
    hp3 = _attn64(qkv2, qkv2, qkv2, dinv, owt, obr, w3t)

    parts3 = _conv64(hp3, src_p, dst_p, w_p, z64)
    return _final(parts3, hp3, dinv, b3r)


# speedup vs baseline: 5.2051x; 1.2139x over previous
import functools

import jax
import jax.numpy as jnp
import numpy as np
from jax import lax
from jax.experimental import pallas as pl
from jax.experimental.pallas import tpu as pltpu
from jax.experimental.pallas import tpu_sc as plsc

_N = 10000
_E = 320000
_DH = 128
_DOUT = 64
_H = 4
_DHEAD = 32

_NC, _NS, _L = 2, 16, 16
_NW = _NC * _NS
_CHUNK = 128
_EPT = 10240
_EPAD = _NW * _EPT
_NCHUNK = _EPT // _CHUNK
_NACC = 10240
_RPT = _NACC // _NS

_BN = 1000
_BQ = 400
_BK = 1000
_F32 = jnp.float32


def _sc_mesh():
    return plsc.VectorSubcoreMesh(
        core_axis_name="c", subcore_axis_name="s",
        num_cores=_NC, num_subcores=_NS)


def _make_sc_conv(d):
    @functools.partial(
        pl.kernel,
        mesh=_sc_mesh(),
        out_type=jax.ShapeDtypeStruct((_NC, _NACC, d), _F32),
        scratch_types=[
            pltpu.VMEM((_CHUNK,), jnp.int32),
            pltpu.VMEM((_CHUNK,), jnp.int32),
            pltpu.VMEM((_CHUNK,), jnp.int32),
            pltpu.VMEM((_CHUNK,), jnp.int32),
            pltpu.VMEM((_CHUNK,), _F32),
            pltpu.VMEM((_CHUNK,), _F32),
            pltpu.VMEM((_CHUNK, d), _F32),
            pltpu.VMEM((_CHUNK, d), _F32),
            pltpu.VMEM_SHARED((_NACC, d), _F32),
            pltpu.SemaphoreType.DMA,
            pltpu.SemaphoreType.DMA,
            pltpu.SemaphoreType.DMA,
            pltpu.SemaphoreType.DMA,
        ],
        name=f"sc_gcn_scatter_d{d}",
        compiler_params=pltpu.CompilerParams(use_tc_tiling_on_sc=(d % 128 == 0)),
    )
    def conv(h_hbm, src_hbm, dst_hbm, w_hbm, zeros_hbm, out_hbm,
             src0, src1, dst0, dst1, w0, w1, rows0, rows1, acc,
             si0, si1, sg0, sg1):
        cid = lax.axis_index("c")
        sid = lax.axis_index("s")
        wid = sid * _NC + cid
        r0 = sid * _RPT
        pltpu.sync_copy(zeros_hbm.at[pl.ds(r0, _RPT)], acc.at[pl.ds(r0, _RPT)])
        plsc.subcore_barrier()
        base = wid * _EPT
        srcs, dsts, ws = (src0, src1), (dst0, dst1), (w0, w1)
        rows, si, sg = (rows0, rows1), (si0, si1), (sg0, sg1)

        def issue_idx(c, p):
            off = base + c * _CHUNK
            pltpu.async_copy(src_hbm.at[pl.ds(off, _CHUNK)], srcs[p], si[p])
            pltpu.async_copy(dst_hbm.at[pl.ds(off, _CHUNK)], dsts[p], si[p])
            pltpu.async_copy(w_hbm.at[pl.ds(off, _CHUNK)], ws[p], si[p])

        def wait_idx(c, p):
            off = base + c * _CHUNK
            pltpu.make_async_copy(src_hbm.at[pl.ds(off, _CHUNK)], srcs[p],
                                  si[p]).wait()
            pltpu.make_async_copy(dst_hbm.at[pl.ds(off, _CHUNK)], dsts[p],
                                  si[p]).wait()
            pltpu.make_async_copy(w_hbm.at[pl.ds(off, _CHUNK)], ws[p],
                                  si[p]).wait()

        def scale(p):
            def grp(g, cc):
                wv = ws[p][pl.ds(g * _L, _L)]
                for i in range(_L):
                    nv = jnp.zeros((_L,), _F32) + wv[i]
                    r = g * _L + i
                    for j in range(d // _L):
                        sl = pl.ds(j * _L, _L)
                        rows[p][r, sl] = rows[p][r, sl] * nv
                return cc

            lax.fori_loop(0, _CHUNK // _L, grp, 0, unroll=False)

        issue_idx(0, 0)
        issue_idx(1, 1)
        wait_idx(0, 0)
        pltpu.async_copy(h_hbm.at[srcs[0]], rows[0], sg[0])

        def pair(g, carry):
            for p in (0, 1):
                c = 2 * g + p
                q = 1 - p

                @pl.when(c < _NCHUNK - 1)
                def _():
                    wait_idx(c + 1, q)
                    pltpu.async_copy(h_hbm.at[srcs[q]], rows[q], sg[q])

                pltpu.make_async_copy(h_hbm.at[srcs[p]], rows[p],
                                      sg[p]).wait()
                scale(p)
                pltpu.sync_copy(rows[p], acc.at[dsts[p]], add=True)

                @pl.when(c < _NCHUNK - 2)
                def _():
                    issue_idx(c + 2, p)

            return carry

        lax.fori_loop(0, _NCHUNK // 2, pair, 0, unroll=False)
        plsc.subcore_barrier()
        pltpu.sync_copy(acc.at[pl.ds(r0, _RPT)],
                        out_hbm.at[cid, pl.ds(r0, _RPT)])

    return conv


@functools.partial(
    pl.kernel,
    mesh=_sc_mesh(),
    out_type=jax.ShapeDtypeStruct((_NC, _NACC), _F32),
    scratch_types=[
        pltpu.VMEM((_NCHUNK, _CHUNK), jnp.int32),
        pltpu.VMEM((_NCHUNK, _CHUNK), _F32),
        pltpu.VMEM_SHARED((_NACC,), _F32),
        pltpu.SemaphoreType.DMA,
    ],
    name="sc_gcn_degree",
)
def _sc_degree(dst_hbm, w_hbm, zeros_hbm, out_hbm, dstall, wall, acc, sem):
    cid = lax.axis_index("c")
    sid = lax.axis_index("s")
    wid = sid * _NC + cid
    r0 = sid * _RPT
    pltpu.sync_copy(zeros_hbm.at[pl.ds(r0, _RPT)], acc.at[pl.ds(r0, _RPT)])
    pltpu.sync_copy(dst_hbm.at[wid], dstall)
    pltpu.sync_copy(w_hbm.at[wid], wall)
    plsc.subcore_barrier()

    def batch(b, carry):
        for k in range(8):
            c = b * 8 + k
            pltpu.async_copy(wall.at[c], acc.at[dstall.at[c]], sem, add=True)
        for k in range(8):
            c = b * 8 + k
            pltpu.make_async_copy(wall.at[c], acc.at[dstall.at[c]],
                                  sem).wait()
        return carry

    lax.fori_loop(0, _NCHUNK // 8, batch, 0, unroll=False)
    plsc.subcore_barrier()
    pltpu.sync_copy(acc.at[pl.ds(r0, _RPT)], out_hbm.at[cid, pl.ds(r0, _RPT)])


def _dinv_body(dp_ref, dinv_ref):
    dp = dp_ref[...]
    dinv_ref[...] = lax.rsqrt(dp[0] + dp[1] + 1.0)


_dinv = pl.pallas_call(
    _dinv_body,
    grid=(1,),
    in_specs=[pl.BlockSpec((2, _NACC // 128, 128), lambda i: (0, 0, 0))],
    out_specs=pl.BlockSpec((_NACC // 128, 128), lambda i: (0, 0)),
    out_shape=jax.ShapeDtypeStruct((_NACC // 128, 128), _F32),
)


def _prep_body(dinv_ref, nf_ref, w1t_ref, hp_ref):
    hp_ref[...] = dinv_ref[...] * jnp.dot(nf_ref[...], w1t_ref[...],
                                          preferred_element_type=_F32)


_prep = pl.pallas_call(
    _prep_body,
    grid=(_N // _BN,),
    in_specs=[
        pl.BlockSpec((_BN, 1), lambda i: (i, 0)),
        pl.BlockSpec((_BN, _DH), lambda i: (i, 0)),
        pl.BlockSpec((_DH, _DH), lambda i: (0, 0)),
    ],
    out_specs=pl.BlockSpec((_BN, _DH), lambda i: (i, 0)),
    out_shape=jax.ShapeDtypeStruct((_N, _DH), _F32),
)


def _qkv_body(parts_ref, hp_ref, dinv_ref, bc_ref, wt_ref, inb_ref, qkv_ref):
    p = parts_ref[...]
    x = dinv_ref[...] * (p[0] + p[1] + hp_ref[...]) + bc_ref[...]
    qkv_ref[...] = jnp.dot(x, wt_ref[...],
                           preferred_element_type=_F32) + inb_ref[...]


_qkv = pl.pallas_call(
    _qkv_body,
    grid=(_N // _BN,),
    in_specs=[
        pl.BlockSpec((2, _BN, _DH), lambda i: (0, i, 0)),
        pl.BlockSpec((_BN, _DH), lambda i: (i, 0)),
        pl.BlockSpec((_BN, 1), lambda i: (i, 0)),
        pl.BlockSpec((1, _DH), lambda i: (0, 0)),
        pl.BlockSpec((_DH, 3 * _DH), lambda i: (0, 0)),
        pl.BlockSpec((1, 3 * _DH), lambda i: (0, 0)),
    ],
    out_specs=pl.BlockSpec((_BN, 3 * _DH), lambda i: (i, 0)),
    out_shape=jax.ShapeDtypeStruct((_N, 3 * _DH), _F32),
)


def _attn_body(dnext, q_ref, k_ref, v_ref, dinv_ref, owt_ref, ob_ref,
               wnt_ref, out_ref):
    q = q_ref[...]
    scale = np.float32(1.0 / np.sqrt(_DHEAD))
    neg = np.float32(-1e30)
    ms = [jnp.full((_BQ, 1), neg, _F32) for _ in range(_H)]
    ls = [jnp.zeros((_BQ, 1), _F32) for _ in range(_H)]
    accs = [jnp.zeros((_BQ, _DHEAD), _F32) for _ in range(_H)]

    def kb_body(kb, carry):
        ms, ls, accs = carry
        kblk = k_ref[pl.ds(kb * _BK, _BK), :]
        vblk = v_ref[pl.ds(kb * _BK, _BK), :]
        nms, nls, naccs = [], [], []
        for h in range(_H):
            hs = slice(h * _DHEAD, (h + 1) * _DHEAD)
            qh = q[:, hs] * scale
            s = lax.dot_general(qh, kblk[:, hs], (((1,), (1,)), ((), ())),
                                preferred_element_type=_F32)
            m_new = jnp.maximum(ms[h], jnp.max(s, axis=1, keepdims=True))
            p = jnp.exp(s - m_new)
            corr = jnp.exp(ms[h] - m_new)
            l_new = ls[h] * corr + jnp.sum(p, axis=1, keepdims=True)
            a_new = accs[h] * corr + lax.dot_general(
                p, vblk[:, hs], (((1,), (0,)), ((), ())),
                preferred_element_type=_F32)
            nms.append(m_new)
            nls.append(l_new)
            naccs.append(a_new)
        return tuple(nms), tuple(nls), tuple(naccs)

    ms, ls, accs = lax.fori_loop(0, _N // _BK, kb_body,
                                 (tuple(ms), tuple(ls), tuple(accs)))
    o = jnp.concatenate([accs[h] / ls[h] for h in range(_H)], axis=1)
    t = jnp.maximum(jnp.dot(o, owt_ref[...], preferred_element_type=_F32)
                    + ob_ref[...], 0.0)
    out_ref[...] = dinv_ref[...] * jnp.dot(t, wnt_ref[...],
                                           preferred_element_type=_F32)


def _make_attn(dnext):
    return pl.pallas_call(
        functools.partial(_attn_body, dnext),
        grid=(_N // _BQ,),
        in_specs=[
            pl.BlockSpec((_BQ, _DH), lambda i: (i, 0)),
            pl.BlockSpec((_N, _DH), lambda i: (0, 1)),
            pl.BlockSpec((_N, _DH), lambda i: (0, 2)),
            pl.BlockSpec((_BQ, 1), lambda i: (i, 0)),
            pl.BlockSpec((_DH, _DH), lambda i: (0, 0)),
            pl.BlockSpec((1, _DH), lambda i: (0, 0)),
            pl.BlockSpec((_DH, dnext), lambda i: (0, 0)),
        ],
        out_specs=pl.BlockSpec((_BQ, dnext), lambda i: (i, 0)),
        out_shape=jax.ShapeDtypeStruct((_N, dnext), _F32),
    )


def _final_body(parts_ref, hp_ref, dinv_ref, b3_ref, out_ref):
    p = parts_ref[...]
    z = dinv_ref[...] * (p[0] + p[1] + hp_ref[...]) + b3_ref[...]
    out_ref[...] = jax.nn.sigmoid(z)


_final = pl.pallas_call(
    _final_body,
    grid=(_N // _BN,),
    in_specs=[
        pl.BlockSpec((2, _BN, _DOUT), lambda i: (0, i, 0)),
        pl.BlockSpec((_BN, _DOUT), lambda i: (i, 0)),
        pl.BlockSpec((_BN, 1), lambda i: (i, 0)),
        pl.BlockSpec((1, _DOUT), lambda i: (0, 0)),
    ],
    out_specs=pl.BlockSpec((_BN, _DOUT), lambda i: (i, 0)),
    out_shape=jax.ShapeDtypeStruct((_N, _DOUT), _F32),
)

_conv128 = _make_sc_conv(_DH)
_conv64 = _make_sc_conv(_DOUT)
_attn128 = _make_attn(_DH)
_attn64 = _make_attn(_DOUT)


def kernel(node_features, edge_index, edge_weight, W1, b1, W2, b2, W3, b3,
           in_proj_w, in_proj_b, out_proj_w, out_proj_b):
    src = edge_index[0]
    dst = edge_index[1]
    pad = _EPAD - _E
    zi = jnp.zeros((pad,), jnp.int32)
    src_p = jnp.concatenate([src, zi])
    dst_p = jnp.concatenate([dst, zi])
    w_p = jnp.concatenate([edge_weight, jnp.zeros((pad,), _F32)])
    dst3 = dst_p.reshape(_NW, _NCHUNK, _CHUNK)
    w3 = w_p.reshape(_NW, _NCHUNK, _CHUNK)

    w1t = W1.T
    w2t = W2.T
    w3t = W3.T
    inwt = in_proj_w.T
    owt = out_proj_w.T
    b1r = b1.reshape(1, -1)
    b2r = b2.reshape(1, -1)
    b3r = b3.reshape(1, -1)
    inbr = in_proj_b.reshape(1, -1)
    obr = out_proj_b.reshape(1, -1)

    z1 = jnp.zeros((_NACC,), _F32)
    z128 = jnp.zeros((_NACC, _DH), _F32)
    z64 = jnp.zeros((_NACC, _DOUT), _F32)

    degp = _sc_degree(dst3, w3, z1)
    dinv_lm = _dinv(degp.reshape(_NC, _NACC // 128, 128))
    dinv = dinv_lm.reshape(_NACC, 1)[:_N]
    hp1 = _prep(dinv, node_features, w1t)

    parts1 = _conv128(hp1, src_p, dst_p, w_p, z128)
    qkv1 = _qkv(parts1, hp1, dinv, b1r, inwt, inbr)
    hp2 = _attn128(qkv1, qkv1, qkv1, dinv, owt, obr, w2t)

    parts2 = _conv128(hp2, src_p, dst_p, w_p, z128)
    qkv2 = _qkv(parts2, hp2, dinv, b2r, inwt, inbr)
    hp3 = _attn64(qkv2, qkv2, qkv2, dinv, owt, obr, w3t)

    parts3 = _conv64(hp3, src_p, dst_p, w_p, z64)
    return _final(parts3, hp3, dinv, b3r)
